# baseline (speedup 1.0000x reference)
import jax
import jax.numpy as jnp
import numpy as np
from jax.experimental import pallas as pl
from jax.experimental.pallas import tpu as pltpu

_B = 16384
_OBS = 256
_OUT = 64
_H = 512
_K = 10
_TEST = 100
_BT = 2048
_NT = _B // _BT
_LOG2PI = float(np.log(2.0 * np.pi))


def _memo_body(x_ref, dx_ref, a_ref,
               ve_w1, ve_b1, ve_w2, ve_b2, pre_w, pre_b,
               emb_ref, post_w, post_b,
               vd_w1, vd_b1, vd_w2, vd_b2,
               d_w1_ref, d_b1, d_w2, d_b2, d_w3, d_b3, d_w4, d_b4,
               ls_ref,
               reconl_ref, prop_ref, sr_ref, sq_ref, x_out_ref):
    dx = dx_ref[...]
    x = x_ref[...]
    x_out_ref[...] = x
    h = jnp.tanh(jnp.dot(dx, ve_w1[...],
                         preferred_element_type=jnp.float32) + ve_b1[...])
    enc0 = jnp.dot(h, ve_w2[...],
                   preferred_element_type=jnp.float32) + ve_b2[...]
    enc = jnp.dot(enc0, pre_w[...],
                  preferred_element_type=jnp.float32) + pre_b[...]
    emb = emb_ref[...]
    emb_sq = jnp.sum(emb * emb, axis=1)
    score = emb_sq - 2.0 * jax.lax.dot_general(
        enc, emb, (((1,), (1,)), ((), ())),
        preferred_element_type=jnp.float32)
    mind = jnp.min(score, axis=1, keepdims=True)
    idxr = jax.lax.broadcasted_iota(jnp.int32, (_BT, _K), 1)
    prop2d = jnp.min(jnp.where(score == mind, idxr, _K), axis=1,
                     keepdims=True)
    onehot = (idxr == prop2d).astype(jnp.float32)
    quant = jnp.dot(onehot, emb, preferred_element_type=jnp.float32)
    postq = jnp.dot(quant, post_w[...],
                    preferred_element_type=jnp.float32) + post_b[...]
    t1 = jnp.tanh(jnp.dot(postq, vd_w1[...],
                          preferred_element_type=jnp.float32) + vd_b1[...])
    recon = jnp.tanh(jnp.dot(t1, vd_w2[...],
                             preferred_element_type=jnp.float32) + vd_b2[...])
    propf = prop2d.astype(jnp.float32)
    h1 = jax.nn.relu(jnp.dot(x, d_w1_ref[:_OBS, :],
                             preferred_element_type=jnp.float32)
                     + propf * d_w1_ref[_OBS:, :] + d_b1[...])
    h2 = jax.nn.relu(jnp.dot(h1, d_w2[...],
                             preferred_element_type=jnp.float32) + d_b2[...])
    h3 = jnp.tanh(jax.nn.relu(jnp.dot(h2, d_w3[...],
                                      preferred_element_type=jnp.float32)
                              + d_b3[...]))
    mu = jnp.dot(h3, d_w4[...],
                 preferred_element_type=jnp.float32) + d_b4[...]
    ls = ls_ref[...]
    z = (a_ref[...] - mu) * jnp.exp(-ls)
    rl_const = jnp.sum(ls) + _OUT * 0.5 * _LOG2PI
    ones_out = jnp.ones((_OUT, 1), jnp.float32)
    rl2d = jnp.dot(0.5 * (z * z), ones_out,
                   preferred_element_type=jnp.float32) + rl_const
    reconl_ref[...] = rl2d
    prop_ref[...] = prop2d
    dr = dx - recon
    dq = enc - quant
    ones_row = jnp.ones((1, _BT), jnp.float32)
    pr = jnp.dot(ones_row, dr * dr, preferred_element_type=jnp.float32)
    pq = jnp.dot(ones_row, dq * dq, preferred_element_type=jnp.float32)
    sr_ref[...] = jnp.sum(pr).reshape(1, 1, 1) + jnp.zeros((1, 1, 128))
    sq_ref[...] = jnp.sum(pq).reshape(1, 1, 1) + jnp.zeros((1, 1, 128))


def _final_body(reconl_ref, sr_ref, sq_ref, loss_ref, vqt_ref):
    tot = (jnp.sum(sr_ref[...]) + 2.0 * jnp.sum(sq_ref[...])) / 128.0
    vq_total = tot * (1.0 / (_B * _OBS))
    vqt_ref[...] = jnp.full((1, 128), vq_total, jnp.float32)
    loss_ref[...] = reconl_ref[...] * vq_total


def _tile_map(i):
    return (i, 0)


def _whole(i):
    return (0, 0)


def kernel(X, Delta_X, A, context_sample, con_dim, ve_w1, ve_b1, ve_w2, ve_b2,
           pre_w, pre_b, emb, post_w, post_b, vd_w1, vd_b1, vd_w2, vd_b2,
           d_w1, d_b1, d_w2, d_b2, d_w3, d_b3, d_w4, d_b4, log_std):
    def row(v):
        return v.reshape(1, -1)

    ins = (X, Delta_X, A,
           ve_w1, row(ve_b1), ve_w2, row(ve_b2), pre_w, row(pre_b),
           emb, post_w, row(post_b),
           vd_w1, row(vd_b1), vd_w2, row(vd_b2),
           d_w1, row(d_b1), d_w2, row(d_b2), d_w3, row(d_b3),
           d_w4, row(d_b4), row(log_std))

    in_specs = [
        pl.BlockSpec((_BT, _OBS), _tile_map),
        pl.BlockSpec((_BT, _OBS), _tile_map),
        pl.BlockSpec((_BT, _OUT), _tile_map),
    ] + [pl.BlockSpec(v.shape, _whole) for v in ins[3:]]

    recon2, prop2, sr, sq, x_out = pl.pallas_call(
        _memo_body,
        grid=(_NT,),
        in_specs=in_specs,
        out_specs=(
            pl.BlockSpec((_BT, 1), _tile_map),
            pl.BlockSpec((_BT, 1), _tile_map),
            pl.BlockSpec((1, 1, 128), lambda i: (i, 0, 0)),
            pl.BlockSpec((1, 1, 128), lambda i: (i, 0, 0)),
            pl.BlockSpec((_BT, _OBS), _tile_map),
        ),
        out_shape=(
            jax.ShapeDtypeStruct((_B, 1), jnp.float32),
            jax.ShapeDtypeStruct((_B, 1), jnp.int32),
            jax.ShapeDtypeStruct((_NT, 1, 128), jnp.float32),
            jax.ShapeDtypeStruct((_NT, 1, 128), jnp.float32),
            jax.ShapeDtypeStruct((_B, _OBS), jnp.float32),
        ),
        compiler_params=pltpu.CompilerParams(
            dimension_semantics=("parallel",)),
    )(*ins)

    loss2, vqt = pl.pallas_call(
        _final_body,
        in_specs=[
            pl.BlockSpec((128, 128), lambda: (0, 0)),
            pl.BlockSpec((_NT, 1, 128), lambda: (0, 0, 0)),
            pl.BlockSpec((_NT, 1, 128), lambda: (0, 0, 0)),
        ],
        out_specs=(
            pl.BlockSpec((128, 128), lambda: (0, 0)),
            pl.BlockSpec((1, 128), lambda: (0, 0)),
        ),
        out_shape=(
            jax.ShapeDtypeStruct((128, 128), jnp.float32),
            jax.ShapeDtypeStruct((1, 128), jnp.float32),
        ),
    )(recon2.reshape(128, 128), sr, sq)

    return (loss2.reshape(_B), recon2.reshape(_B), x_out, prop2.reshape(_B),
            vqt[0, 0])


# speedup vs baseline: 2.0886x; 2.0886x over previous
import jax
import jax.numpy as jnp
import numpy as np
from jax.experimental import pallas as pl
from jax.experimental.pallas import tpu as pltpu

_B = 16384
_OBS = 256
_OUT = 64
_H = 512
_K = 10
_TEST = 100
_BT = 2048
_NT = _B // _BT
_LOG2PI = float(np.log(2.0 * np.pi))


def _memo_body(x_ref, dx_ref, a_ref,
               ve_w1, ve_b1, ve_w2, ve_b2, pre_w, pre_b,
               embT, emb, emb_sq, post_w, post_b,
               vd_w1, vd_b1, vd_w2, vd_b2,
               d_w1_ref, d_b1, d_w2, d_b2, d_w3, d_b3, d_w4, d_b4,
               ls_ref,
               reconl_ref, prop_ref, sr_ref, sq_ref, x_out_ref):
    dx = dx_ref[...]
    x = x_ref[...]
    x_out_ref[...] = x
    h = jnp.tanh(jnp.dot(dx, ve_w1[...],
                         preferred_element_type=jnp.float32) + ve_b1[...])
    enc0 = jnp.dot(h, ve_w2[...],
                   preferred_element_type=jnp.float32) + ve_b2[...]
    enc = jnp.dot(enc0, pre_w[...],
                  preferred_element_type=jnp.float32) + pre_b[...]
    score = emb_sq[...] - 2.0 * jnp.dot(enc, embT[...],
                                        preferred_element_type=jnp.float32)
    mind = jnp.min(score, axis=1, keepdims=True)
    idxr = jax.lax.broadcasted_iota(jnp.int32, (_BT, _K), 1)
    prop2d = jnp.min(jnp.where(score == mind, idxr, _K), axis=1,
                     keepdims=True)
    onehot = (idxr == prop2d).astype(jnp.float32)
    quant = jnp.dot(onehot, emb[...], preferred_element_type=jnp.float32)
    postq = jnp.dot(quant, post_w[...],
                    preferred_element_type=jnp.float32) + post_b[...]
    t1 = jnp.tanh(jnp.dot(postq, vd_w1[...],
                          preferred_element_type=jnp.float32) + vd_b1[...])
    recon = jnp.tanh(jnp.dot(t1, vd_w2[...],
                             preferred_element_type=jnp.float32) + vd_b2[...])
    propf = prop2d.astype(jnp.float32)
    h1 = jax.nn.relu(jnp.dot(x, d_w1_ref[:_OBS, :],
                             preferred_element_type=jnp.float32)
                     + propf * d_w1_ref[_OBS:, :] + d_b1[...])
    h2 = jax.nn.relu(jnp.dot(h1, d_w2[...],
                             preferred_element_type=jnp.float32) + d_b2[...])
    h3 = jnp.tanh(jax.nn.relu(jnp.dot(h2, d_w3[...],
                                      preferred_element_type=jnp.float32)
                              + d_b3[...]))
    mu = jnp.dot(h3, d_w4[...],
                 preferred_element_type=jnp.float32) + d_b4[...]
    ls = ls_ref[...]
    z = (a_ref[...] - mu) * jnp.exp(-ls)
    rl_const = jnp.sum(ls) + _OUT * 0.5 * _LOG2PI
    ones_out = jnp.ones((_OUT, 1), jnp.float32)
    rl2d = jnp.dot(0.5 * (z * z), ones_out,
                   preferred_element_type=jnp.float32) + rl_const
    reconl_ref[...] = rl2d
    prop_ref[...] = prop2d
    dr = dx - recon
    dq = enc - quant
    ones_row = jnp.ones((1, _BT), jnp.float32)
    pr = jnp.dot(ones_row, dr * dr, preferred_element_type=jnp.float32)
    pq = jnp.dot(ones_row, dq * dq, preferred_element_type=jnp.float32)
    sr_ref[...] = jnp.sum(pr).reshape(1, 1, 1) + jnp.zeros((1, 1, 128))
    sq_ref[...] = jnp.sum(pq).reshape(1, 1, 1) + jnp.zeros((1, 1, 128))


def _final_body(reconl_ref, sr_ref, sq_ref, loss_ref, vqt_ref):
    tot = (jnp.sum(sr_ref[...]) + 2.0 * jnp.sum(sq_ref[...])) / 128.0
    vq_total = tot * (1.0 / (_B * _OBS))
    vqt_ref[...] = jnp.full((1, 128), vq_total, jnp.float32)
    loss_ref[...] = reconl_ref[...] * vq_total


def _tile_map(i):
    return (i, 0)


def _whole(i):
    return (0, 0)


def kernel(X, Delta_X, A, context_sample, con_dim, ve_w1, ve_b1, ve_w2, ve_b2,
           pre_w, pre_b, emb, post_w, post_b, vd_w1, vd_b1, vd_w2, vd_b2,
           d_w1, d_b1, d_w2, d_b2, d_w3, d_b3, d_w4, d_b4, log_std):
    embT = emb.T
    emb_sq = jnp.sum(emb * emb, axis=1)[None, :]

    def row(v):
        return v.reshape(1, -1)

    ins = (X, Delta_X, A,
           ve_w1, row(ve_b1), ve_w2, row(ve_b2), pre_w, row(pre_b),
           embT, emb, emb_sq, post_w, row(post_b),
           vd_w1, row(vd_b1), vd_w2, row(vd_b2),
           d_w1, row(d_b1), d_w2, row(d_b2), d_w3, row(d_b3),
           d_w4, row(d_b4), row(log_std))

    in_specs = [
        pl.BlockSpec((_BT, _OBS), _tile_map),
        pl.BlockSpec((_BT, _OBS), _tile_map),
        pl.BlockSpec((_BT, _OUT), _tile_map),
    ] + [pl.BlockSpec(v.shape, _whole) for v in ins[3:]]

    recon2, prop2, sr, sq, x_out = pl.pallas_call(
        _memo_body,
        grid=(_NT,),
        in_specs=in_specs,
        out_specs=(
            pl.BlockSpec((_BT, 1), _tile_map),
            pl.BlockSpec((_BT, 1), _tile_map),
            pl.BlockSpec((1, 1, 128), lambda i: (i, 0, 0)),
            pl.BlockSpec((1, 1, 128), lambda i: (i, 0, 0)),
            pl.BlockSpec((_BT, _OBS), _tile_map),
        ),
        out_shape=(
            jax.ShapeDtypeStruct((_B, 1), jnp.float32),
            jax.ShapeDtypeStruct((_B, 1), jnp.int32),
            jax.ShapeDtypeStruct((_NT, 1, 128), jnp.float32),
            jax.ShapeDtypeStruct((_NT, 1, 128), jnp.float32),
            jax.ShapeDtypeStruct((_B, _OBS), jnp.float32),
        ),
        compiler_params=pltpu.CompilerParams(
            dimension_semantics=("parallel",)),
    )(*ins)

    loss2, vqt = pl.pallas_call(
        _final_body,
        in_specs=[
            pl.BlockSpec((128, 128), lambda: (0, 0)),
            pl.BlockSpec((_NT, 1, 128), lambda: (0, 0, 0)),
            pl.BlockSpec((_NT, 1, 128), lambda: (0, 0, 0)),
        ],
        out_specs=(
            pl.BlockSpec((128, 128), lambda: (0, 0)),
            pl.BlockSpec((1, 128), lambda: (0, 0)),
        ),
        out_shape=(
            jax.ShapeDtypeStruct((128, 128), jnp.float32),
            jax.ShapeDtypeStruct((1, 128), jnp.float32),
        ),
    )(recon2.reshape(128, 128), sr, sq)

    return (loss2.reshape(_B), recon2.reshape(_B), x_out, prop2.reshape(_B),
            vqt[0, 0])


# speedup vs baseline: 2.0969x; 1.0039x over previous
import jax
import jax.numpy as jnp
import numpy as np
from jax.experimental import pallas as pl
from jax.experimental.pallas import tpu as pltpu

_B = 16384
_OBS = 256
_OUT = 64
_H = 512
_K = 10
_TEST = 100
_BT = 2048
_NT = _B // _BT
_LOG2PI = float(np.log(2.0 * np.pi))


def _memo_body(x_ref, dx_ref, a_ref,
               ve_w1, ve_b1, ve_w2, ve_b2, pre_w, pre_b,
               embT, emb, emb_sq, post_w, post_b,
               vd_w1, vd_b1, vd_w2, vd_b2,
               d_w1_ref, d_b1, d_w2, d_b2, d_w3, d_b3, d_w4, d_b4,
               ls_ref,
               reconl_ref, prop_ref, sr_ref, sq_ref, x_out_ref):
    dx = dx_ref[...]
    x = x_ref[...]
    x_out_ref[...] = x
    h = jnp.tanh(jnp.dot(dx, ve_w1[...],
                         preferred_element_type=jnp.float32) + ve_b1[...])
    enc0 = jnp.dot(h, ve_w2[...],
                   preferred_element_type=jnp.float32) + ve_b2[...]
    enc = jnp.dot(enc0, pre_w[...],
                  preferred_element_type=jnp.float32) + pre_b[...]
    score = emb_sq[...] - 2.0 * jnp.dot(enc, embT[...],
                                        preferred_element_type=jnp.float32)
    mind = jnp.min(score, axis=1, keepdims=True)
    idxr = jax.lax.broadcasted_iota(jnp.int32, (_BT, _K), 1)
    prop2d = jnp.min(jnp.where(score == mind, idxr, _K), axis=1,
                     keepdims=True)
    onehot = (idxr == prop2d).astype(jnp.float32)
    quant = jnp.dot(onehot, emb[...], preferred_element_type=jnp.float32)
    postq = jnp.dot(quant, post_w[...],
                    preferred_element_type=jnp.float32) + post_b[...]
    t1 = jnp.tanh(jnp.dot(postq, vd_w1[...],
                          preferred_element_type=jnp.float32) + vd_b1[...])
    recon = jnp.tanh(jnp.dot(t1, vd_w2[...],
                             preferred_element_type=jnp.float32) + vd_b2[...])
    propf = prop2d.astype(jnp.float32)
    h1 = jax.nn.relu(jnp.dot(x, d_w1_ref[:_OBS, :],
                             preferred_element_type=jnp.float32)
                     + propf * d_w1_ref[_OBS:, :] + d_b1[...])
    h2 = jax.nn.relu(jnp.dot(h1, d_w2[...],
                             preferred_element_type=jnp.float32) + d_b2[...])
    h3 = jnp.tanh(jax.nn.relu(jnp.dot(h2, d_w3[...],
                                      preferred_element_type=jnp.float32)
                              + d_b3[...]))
    mu = jnp.dot(h3, d_w4[...],
                 preferred_element_type=jnp.float32) + d_b4[...]
    ls = ls_ref[...]
    z = (a_ref[...] - mu) * jnp.exp(-ls)
    rl_const = jnp.sum(ls) + _OUT * 0.5 * _LOG2PI
    ones_out = jnp.ones((_OUT, 1), jnp.float32)
    rl2d = jnp.dot(0.5 * (z * z), ones_out,
                   preferred_element_type=jnp.float32) + rl_const
    reconl_ref[...] = rl2d
    prop_ref[...] = prop2d
    dr = dx - recon
    dq = enc - quant
    ones_row = jnp.ones((1, _BT), jnp.float32)
    pr = jnp.dot(ones_row, dr * dr, preferred_element_type=jnp.float32)
    pq = jnp.dot(ones_row, dq * dq, preferred_element_type=jnp.float32)
    sr_ref[...] = jnp.sum(pr).reshape(1, 1, 1) + jnp.zeros((1, 1, 128))
    sq_ref[...] = jnp.sum(pq).reshape(1, 1, 1) + jnp.zeros((1, 1, 128))


def _final_body(reconl_ref, sr_ref, sq_ref, loss_ref, vqt_ref):
    tot = (jnp.sum(sr_ref[...]) + 2.0 * jnp.sum(sq_ref[...])) / 128.0
    vq_total = tot * (1.0 / (_B * _OBS))
    vqt_ref[...] = jnp.full((1, 128), vq_total, jnp.float32)
    loss_ref[...] = reconl_ref[...] * vq_total


def _tile_map(i):
    return (i, 0)


def _whole(i):
    return (0, 0)


def kernel(X, Delta_X, A, context_sample, con_dim, ve_w1, ve_b1, ve_w2, ve_b2,
           pre_w, pre_b, emb, post_w, post_b, vd_w1, vd_b1, vd_w2, vd_b2,
           d_w1, d_b1, d_w2, d_b2, d_w3, d_b3, d_w4, d_b4, log_std):
    embT = emb.T
    emb_sq = jnp.sum(emb * emb, axis=1)[None, :]

    ins = (X, Delta_X, A,
           ve_w1, ve_b1, ve_w2, ve_b2, pre_w, pre_b,
           embT, emb, emb_sq, post_w, post_b,
           vd_w1, vd_b1, vd_w2, vd_b2,
           d_w1, d_b1, d_w2, d_b2, d_w3, d_b3,
           d_w4, d_b4, log_std)

    in_specs = [
        pl.BlockSpec((_BT, _OBS), _tile_map),
        pl.BlockSpec((_BT, _OBS), _tile_map),
        pl.BlockSpec((_BT, _OUT), _tile_map),
    ] + [pl.BlockSpec(v.shape, lambda i, n=v.ndim: (0,) * n)
         for v in ins[3:]]

    recon2, prop2, sr, sq, x_out = pl.pallas_call(
        _memo_body,
        grid=(_NT,),
        in_specs=in_specs,
        out_specs=(
            pl.BlockSpec((_BT, 1), _tile_map),
            pl.BlockSpec((_BT, 1), _tile_map),
            pl.BlockSpec((1, 1, 128), lambda i: (i, 0, 0)),
            pl.BlockSpec((1, 1, 128), lambda i: (i, 0, 0)),
            pl.BlockSpec((_BT, _OBS), _tile_map),
        ),
        out_shape=(
            jax.ShapeDtypeStruct((_B, 1), jnp.float32),
            jax.ShapeDtypeStruct((_B, 1), jnp.int32),
            jax.ShapeDtypeStruct((_NT, 1, 128), jnp.float32),
            jax.ShapeDtypeStruct((_NT, 1, 128), jnp.float32),
            jax.ShapeDtypeStruct((_B, _OBS), jnp.float32),
        ),
        compiler_params=pltpu.CompilerParams(
            dimension_semantics=("parallel",)),
    )(*ins)

    loss2, vqt = pl.pallas_call(
        _final_body,
        in_specs=[
            pl.BlockSpec((128, 128), lambda: (0, 0)),
            pl.BlockSpec((_NT, 1, 128), lambda: (0, 0, 0)),
            pl.BlockSpec((_NT, 1, 128), lambda: (0, 0, 0)),
        ],
        out_specs=(
            pl.BlockSpec((128, 128), lambda: (0, 0)),
            pl.BlockSpec((1, 128), lambda: (0, 0)),
        ),
        out_shape=(
            jax.ShapeDtypeStruct((128, 128), jnp.float32),
            jax.ShapeDtypeStruct((1, 128), jnp.float32),
        ),
    )(recon2.reshape(128, 128), sr, sq)

    return (loss2.reshape(_B), recon2.reshape(_B), x_out, prop2.reshape(_B),
            vqt[0, 0])
